# baseline (speedup 1.0000x reference)
import functools

import jax
import jax.numpy as jnp
from jax import lax
from jax.experimental import pallas as pl
from jax.experimental.pallas import tpu as pltpu
from jax.experimental.pallas import tpu_sc as plsc

_NUM_EMBEDDINGS = 1000000
_DIM = 64
_BATCH = 16384

_info = plsc.get_sparse_core_info()
_NC, _NS = _info.num_cores, _info.num_subcores
_NW = _NC * _NS
_B_PER_W = _BATCH // _NW
_CHUNK = 128
_N_CH = _B_PER_W // _CHUNK


def _body(table_hbm, idx_hbm, out_hbm, idx_v, rows_v, sem):
    wid = lax.axis_index("s") * _NC + lax.axis_index("c")
    pltpu.sync_copy(idx_hbm.at[wid], idx_v)
    copies = [
        pltpu.async_copy(table_hbm.at[idx_v.at[j]], rows_v.at[j], sem)
        for j in range(_N_CH)
    ]
    for c in copies:
        c.wait()
    pltpu.sync_copy(rows_v, out_hbm.at[wid])


_gather = functools.partial(
    pl.kernel,
    mesh=plsc.VectorSubcoreMesh(core_axis_name="c", subcore_axis_name="s"),
    out_type=jax.ShapeDtypeStruct((_NW, _N_CH, _CHUNK, _DIM), jnp.float32),
    scratch_types=[
        pltpu.VMEM((_N_CH, _CHUNK), jnp.int32),
        pltpu.VMEM((_N_CH, _CHUNK, _DIM), jnp.float32),
        pltpu.SemaphoreType.DMA,
    ],
    compiler_params=pltpu.CompilerParams(use_tc_tiling_on_sc=False),
)(_body)


@jax.jit
def kernel(token_ids, embedding_weights):
    idx = token_ids.astype(jnp.int32).reshape(_NW, _N_CH, _CHUNK)
    out = _gather(embedding_weights, idx)
    return out.reshape(_BATCH, _DIM)


# speedup vs baseline: 2.6034x; 2.6034x over previous
import functools

import jax
import jax.numpy as jnp
from jax import lax
from jax.experimental import pallas as pl
from jax.experimental.pallas import tpu as pltpu
from jax.experimental.pallas import tpu_sc as plsc

_NUM_EMBEDDINGS = 1000000
_DIM = 64
_BATCH = 16384

_info = plsc.get_sparse_core_info()
_NC, _NS = _info.num_cores, _info.num_subcores
_NW = _NC * _NS
_B_PER_W = _BATCH // _NW
_NBUF = 4

_IOTA16 = None


def _extract_column(block, cols, d, t):
    d_vec = jnp.full((16,), d, dtype=jnp.int32)
    t_vec = jnp.full((16,), t, dtype=jnp.int32)
    base_rows = lax.iota(jnp.int32, 16)
    for g in range(_DIM // 16):
        rows = base_rows + (16 * g)
        vals = plsc.load_gather(block, [rows, d_vec])
        plsc.store_scatter(cols, [rows, t_vec], vals)


def _body(table_t, idx_hbm, out_t, idx_v, blocks, cols, s0, s1, s2, s3):
    sems = [s0, s1, s2, s3]
    wid = lax.axis_index("s") * _NC + lax.axis_index("c")
    base = pl.multiple_of(wid * _B_PER_W, 128)
    pltpu.sync_copy(idx_hbm.at[pl.ds(base, _B_PER_W)], idx_v)

    def fetch_tok(tok, p):
        blk = pl.multiple_of(tok & jnp.int32(-128), 128)
        pltpu.async_copy(
            table_t.at[:, pl.ds(blk, 128)], blocks.at[p], sems[p]
        )

    n_groups = _B_PER_W // 16
    vec0 = idx_v[pl.ds(0, 16)]
    vec1 = idx_v[pl.ds(16, 16)]
    for p in range(_NBUF):
        fetch_tok(vec0[p], p)

    def outer(g, carry):
        cur, nxt = carry
        for l in range(16):
            t = g * 16 + l
            p = l % _NBUF
            pltpu.make_async_copy(
                table_t.at[:, pl.ds(0, 128)], blocks.at[p], sems[p]
            ).wait()
            tok = cur[l]
            d = tok & jnp.int32(127)
            _extract_column(blocks.at[p], cols, d, t)
            ntok = cur[l + _NBUF] if l < 16 - _NBUF else nxt[l - (16 - _NBUF)]

            @pl.when(t + _NBUF < _B_PER_W)
            def _():
                fetch_tok(ntok, p)

        off = jnp.minimum((g + 2) * 16, _B_PER_W - 16)
        new_nxt = idx_v[pl.ds(off, 16)]
        return nxt, new_nxt

    lax.fori_loop(0, n_groups, outer, (vec0, vec1))
    pltpu.sync_copy(cols, out_t.at[:, pl.ds(base, _B_PER_W)])


_gather = functools.partial(
    pl.kernel,
    mesh=plsc.VectorSubcoreMesh(core_axis_name="c", subcore_axis_name="s"),
    out_type=jax.ShapeDtypeStruct((_DIM, _BATCH), jnp.float32),
    scratch_types=[
        pltpu.VMEM((_B_PER_W,), jnp.int32),
        pltpu.VMEM((_NBUF, _DIM, 128), jnp.float32),
        pltpu.VMEM((_DIM, _B_PER_W), jnp.float32),
        pltpu.SemaphoreType.DMA,
        pltpu.SemaphoreType.DMA,
        pltpu.SemaphoreType.DMA,
        pltpu.SemaphoreType.DMA,
    ],
    compiler_params=pltpu.CompilerParams(needs_layout_passes=False),
)(_body)


@jax.jit
def kernel(token_ids, embedding_weights):
    idx = token_ids.astype(jnp.int32)
    out_t = _gather(embedding_weights.T, idx)
    return out_t.T


# speedup vs baseline: 3.0293x; 1.1636x over previous
import functools

import jax
import jax.numpy as jnp
from jax import lax
from jax.experimental import pallas as pl
from jax.experimental.pallas import tpu as pltpu
from jax.experimental.pallas import tpu_sc as plsc

_NUM_EMBEDDINGS = 1000000
_DIM = 64
_BATCH = 16384

_info = plsc.get_sparse_core_info()
_NC, _NS = _info.num_cores, _info.num_subcores
_NW = _NC * _NS
_B_PER_W = _BATCH // _NW
_NBUF = 8

_IOTA16 = None


def _extract_column(block, cols, d, t):
    d_vec = jnp.full((16,), d, dtype=jnp.int32)
    t_vec = jnp.full((16,), t, dtype=jnp.int32)
    base_rows = lax.iota(jnp.int32, 16)
    for g in range(_DIM // 16):
        rows = base_rows + (16 * g)
        vals = plsc.load_gather(block, [rows, d_vec])
        plsc.store_scatter(cols, [rows, t_vec], vals)


def _body(table_t, idx_hbm, out_t, idx_v, blocks, cols, *sems):
    wid = lax.axis_index("s") * _NC + lax.axis_index("c")
    base = pl.multiple_of(wid * _B_PER_W, 128)
    pltpu.sync_copy(idx_hbm.at[pl.ds(base, _B_PER_W)], idx_v)

    def fetch_tok(tok, p):
        blk = pl.multiple_of(tok & jnp.int32(-128), 128)
        pltpu.async_copy(
            table_t.at[:, pl.ds(blk, 128)], blocks.at[p], sems[p]
        )

    n_groups = _B_PER_W // 16
    vec0 = idx_v[pl.ds(0, 16)]
    vec1 = idx_v[pl.ds(16, 16)]
    for p in range(_NBUF):
        fetch_tok(vec0[p], p)

    def outer(g, carry):
        cur, nxt = carry
        for l in range(16):
            t = g * 16 + l
            p = l % _NBUF
            pltpu.make_async_copy(
                table_t.at[:, pl.ds(0, 128)], blocks.at[p], sems[p]
            ).wait()
            tok = cur[l]
            d = tok & jnp.int32(127)
            _extract_column(blocks.at[p], cols, d, t)
            ntok = cur[l + _NBUF] if l < 16 - _NBUF else nxt[l - (16 - _NBUF)]

            @pl.when(t + _NBUF < _B_PER_W)
            def _():
                fetch_tok(ntok, p)

        off = jnp.minimum((g + 2) * 16, _B_PER_W - 16)
        new_nxt = idx_v[pl.ds(off, 16)]
        return nxt, new_nxt

    lax.fori_loop(0, n_groups, outer, (vec0, vec1))
    pltpu.sync_copy(cols, out_t.at[:, pl.ds(base, _B_PER_W)])


_gather = functools.partial(
    pl.kernel,
    mesh=plsc.VectorSubcoreMesh(core_axis_name="c", subcore_axis_name="s"),
    out_type=jax.ShapeDtypeStruct((_DIM, _BATCH), jnp.float32),
    scratch_types=[
        pltpu.VMEM((_B_PER_W,), jnp.int32),
        pltpu.VMEM((_NBUF, _DIM, 128), jnp.float32),
        pltpu.VMEM((_DIM, _B_PER_W), jnp.float32),
    ]
    + [pltpu.SemaphoreType.DMA] * _NBUF,
    compiler_params=pltpu.CompilerParams(needs_layout_passes=False),
)(_body)


@jax.jit
def kernel(token_ids, embedding_weights):
    idx = token_ids.astype(jnp.int32)
    out_t = _gather(embedding_weights.T, idx)
    return out_t.T


# speedup vs baseline: 3.0317x; 1.0008x over previous
import functools

import jax
import jax.numpy as jnp
from jax import lax
from jax.experimental import pallas as pl
from jax.experimental.pallas import tpu as pltpu
from jax.experimental.pallas import tpu_sc as plsc

_NUM_EMBEDDINGS = 1000000
_DIM = 64
_BATCH = 16384

_info = plsc.get_sparse_core_info()
_NC, _NS = _info.num_cores, _info.num_subcores
_NW = _NC * _NS
_B_PER_W = _BATCH // _NW
_NBUF = 8
_W = 128

_IOTA16 = None


def _extract_column(block, cols, d, t):
    d_vec = jnp.full((16,), d, dtype=jnp.int32)
    t_vec = jnp.full((16,), t, dtype=jnp.int32)
    base_rows = lax.iota(jnp.int32, 16)
    for g in range(_DIM // 16):
        rows = base_rows + (16 * g)
        vals = plsc.load_gather(block, [rows, d_vec])
        plsc.store_scatter(cols, [rows, t_vec], vals)


def _body(table_t, idx_hbm, out_t, idx_v, blocks, cols, *sems):
    wid = lax.axis_index("s") * _NC + lax.axis_index("c")
    base = pl.multiple_of(wid * _B_PER_W, 128)
    pltpu.sync_copy(idx_hbm.at[pl.ds(base, _B_PER_W)], idx_v)

    def fetch_tok(tok, p):
        blk = pl.multiple_of(tok & jnp.int32(-_W), _W)
        pltpu.async_copy(
            table_t.at[:, pl.ds(blk, _W)], blocks.at[p], sems[p]
        )

    n_groups = _B_PER_W // 16
    vec0 = idx_v[pl.ds(0, 16)]
    vec1 = idx_v[pl.ds(16, 16)]
    for p in range(_NBUF):
        fetch_tok(vec0[p], p)

    def outer(g, carry):
        cur, nxt = carry
        for l in range(16):
            t = g * 16 + l
            p = l % _NBUF
            pltpu.make_async_copy(
                table_t.at[:, pl.ds(0, _W)], blocks.at[p], sems[p]
            ).wait()
            tok = cur[l]
            d = tok & jnp.int32(_W - 1)
            _extract_column(blocks.at[p], cols, d, t)
            ntok = cur[l + _NBUF] if l < 16 - _NBUF else nxt[l - (16 - _NBUF)]

            @pl.when(t + _NBUF < _B_PER_W)
            def _():
                fetch_tok(ntok, p)

        off = jnp.minimum((g + 2) * 16, _B_PER_W - 16)
        new_nxt = idx_v[pl.ds(off, 16)]
        return nxt, new_nxt

    lax.fori_loop(0, n_groups, outer, (vec0, vec1))
    pltpu.sync_copy(cols, out_t.at[:, pl.ds(base, _B_PER_W)])


_gather = functools.partial(
    pl.kernel,
    mesh=plsc.VectorSubcoreMesh(core_axis_name="c", subcore_axis_name="s"),
    out_type=jax.ShapeDtypeStruct((_DIM, _BATCH), jnp.float32),
    scratch_types=[
        pltpu.VMEM((_B_PER_W,), jnp.int32),
        pltpu.VMEM((_NBUF, _DIM, _W), jnp.float32),
        pltpu.VMEM((_DIM, _B_PER_W), jnp.float32),
    ]
    + [pltpu.SemaphoreType.DMA] * _NBUF,
    compiler_params=pltpu.CompilerParams(needs_layout_passes=False),
)(_body)


@jax.jit
def kernel(token_ids, embedding_weights):
    idx = token_ids.astype(jnp.int32)
    out_t = _gather(embedding_weights.T, idx)
    return out_t.T
